# Initial kernel scaffold; baseline (speedup 1.0000x reference)
import jax
import jax.numpy as jnp
from jax.experimental import pallas as pl
from jax.experimental.pallas import tpu as pltpu

_N = 128
_D = 256


def _greedy_kernel(x1_ref, x2_ref, out_ref):
    sub = jax.lax.dot_general(
        x1_ref[...], x2_ref[...],
        (((1,), (1,)), ((), ())),
        preferred_element_type=jnp.float32,
    )

    row_iota = jax.lax.broadcasted_iota(jnp.int32, (_N, _N), 0)
    col_iota = jax.lax.broadcasted_iota(jnp.int32, (_N, _N), 1)
    flat_iota = row_iota * _N + col_iota
    neg_inf = jnp.float32(-jnp.inf)

    def body(_, carry):
        m, total = carry
        gmax = jnp.max(m)
        idx = jnp.min(jnp.where(m == gmax, flat_iota, jnp.int32(_N * _N)))
        r = idx // _N
        c = idx - r * _N
        m = jnp.where((row_iota == r) | (col_iota == c), neg_inf, m)
        return m, total + gmax

    _, total = jax.lax.fori_loop(0, _N, body, (sub, jnp.float32(0.0)))
    out_ref[0, 0] = jnp.tanh(total / jnp.float32(_N))


def kernel(x1, x2):
    out = pl.pallas_call(
        _greedy_kernel,
        out_shape=jax.ShapeDtypeStruct((1, 1), jnp.float32),
    )(x1, x2)
    return jnp.reshape(out, (1,))


# speedup vs baseline: 19.3033x; 19.3033x over previous
import jax
import jax.numpy as jnp
from jax.experimental import pallas as pl
from jax.experimental.pallas import tpu as pltpu

_N = 128
_D = 256


def _greedy_kernel(x1_ref, x2_ref, out_ref):
    sub = jax.lax.dot_general(
        x1_ref[...], x2_ref[...],
        (((1,), (1,)), ((), ())),
        preferred_element_type=jnp.float32,
    )

    row_iota = jax.lax.broadcasted_iota(jnp.int32, (_N, _N), 0)
    col_iota = jax.lax.broadcasted_iota(jnp.int32, (_N, _N), 1)
    flat_iota = row_iota * _N + col_iota
    neg_inf = jnp.float32(-jnp.inf)

    def body(_, carry):
        m, total = carry
        gmax = jnp.max(m)
        idx = jnp.min(jnp.where(m == gmax, flat_iota, jnp.int32(_N * _N)))
        r = idx // _N
        c = idx - r * _N
        m = jnp.where((row_iota == r) | (col_iota == c), neg_inf, m)
        return m, total + gmax

    _, total = jax.lax.fori_loop(0, _N, body, (sub, jnp.float32(0.0)))
    out_ref[0, 0] = jnp.tanh(total / jnp.float32(_N))


def kernel(x1, x2):
    out = pl.pallas_call(
        _greedy_kernel,
        out_shape=jax.ShapeDtypeStruct((1, 1), jnp.float32),
        out_specs=pl.BlockSpec(memory_space=pltpu.SMEM),
    )(x1, x2)
    return jnp.reshape(out, (1,))


# speedup vs baseline: 20.2556x; 1.0493x over previous
import jax
import jax.numpy as jnp
from jax.experimental import pallas as pl
from jax.experimental.pallas import tpu as pltpu

_N = 128
_G = 16


def _greedy_kernel(x1_ref, x2_ref, out_ref):
    sub = jax.lax.dot_general(
        x1_ref[...], x2_ref[...],
        (((1,), (1,)), ((), ())),
        preferred_element_type=jnp.float32,
    )

    m3_0 = jnp.reshape(sub, (_G, 8, _N))
    lane = jax.lax.broadcasted_iota(jnp.int32, (1, _N), 1)
    sub_iota = jax.lax.broadcasted_iota(jnp.int32, (8, _N), 0)
    g_iota3 = jax.lax.broadcasted_iota(jnp.int32, (_G, 8, _N), 0)
    s_iota3 = jax.lax.broadcasted_iota(jnp.int32, (_G, 8, _N), 1)
    neg_inf = jnp.float32(-jnp.inf)
    big = jnp.int32(1 << 30)

    def body(_, carry):
        m3, col_kill, total = carry
        vv = jnp.reshape(m3, (8, 2, 8, _N))
        a, b = vv[:, 0], vv[:, 1]
        keep = a >= b
        pair = jax.lax.broadcasted_iota(jnp.int32, (8, 8, _N), 0)
        g = jnp.where(keep, pair * 2, pair * 2 + 1)
        v = jnp.where(keep, a, b)
        for n in (4, 2, 1):
            vv = jnp.reshape(v, (n, 2, 8, _N))
            gg = jnp.reshape(g, (n, 2, 8, _N))
            a, b = vv[:, 0], vv[:, 1]
            ga, gb = gg[:, 0], gg[:, 1]
            keep = a >= b
            v = jnp.where(keep, a, b)
            g = jnp.where(keep, ga, gb)
        v1 = v[0]
        row2 = g[0] * 8 + sub_iota
        cm = jnp.max(v1, axis=0, keepdims=True)
        rmin = jnp.min(jnp.where(v1 == cm, row2, big), axis=0, keepdims=True)
        cmm = cm + col_kill
        gmax = jnp.max(cmm)
        key = jnp.min(jnp.where(cmm == gmax, rmin * _N + lane, big))
        g_r = jax.lax.shift_right_logical(key, 10)
        s_r = jax.lax.shift_right_logical(key, 7) & 7
        c = key & (_N - 1)
        m3 = jnp.where((g_iota3 == g_r) & (s_iota3 == s_r), neg_inf, m3)
        col_kill = jnp.where(lane == c, neg_inf, col_kill)
        return m3, col_kill, total + gmax

    init = (m3_0, jnp.zeros((1, _N), jnp.float32), jnp.float32(0.0))
    _, _, total = jax.lax.fori_loop(0, _N, body, init)
    out_ref[0, 0] = jnp.tanh(total / jnp.float32(_N))


def kernel(x1, x2):
    out = pl.pallas_call(
        _greedy_kernel,
        out_shape=jax.ShapeDtypeStruct((1, 1), jnp.float32),
        out_specs=pl.BlockSpec(memory_space=pltpu.SMEM),
    )(x1, x2)
    return jnp.reshape(out, (1,))


# speedup vs baseline: 26.2407x; 1.2955x over previous
import jax
import jax.numpy as jnp
from jax.experimental import pallas as pl
from jax.experimental.pallas import tpu as pltpu

_N = 128
_G = 16


def _greedy_kernel(x1_ref, x2_ref, out_ref):
    sub = jax.lax.dot_general(
        x1_ref[...], x2_ref[...],
        (((1,), (1,)), ((), ())),
        preferred_element_type=jnp.float32,
    )

    m3_0 = jnp.reshape(sub, (_G, 8, _N))
    lane = jax.lax.broadcasted_iota(jnp.int32, (1, _N), 1)
    lane_f = lane.astype(jnp.float32)
    sub_iota = jax.lax.broadcasted_iota(jnp.int32, (8, _N), 0)
    g_iota3 = jax.lax.broadcasted_iota(jnp.int32, (_G, 8, _N), 0)
    s_iota3 = jax.lax.broadcasted_iota(jnp.int32, (_G, 8, _N), 1)
    neg_inf = jnp.float32(-jnp.inf)
    big = jnp.int32(1 << 30)
    big_f = jnp.float32(1e9)

    def lane_allmax(x):
        for s in (64, 32, 16, 8, 4, 2, 1):
            x = jnp.maximum(x, pltpu.roll(x, s, 1))
        return x

    def lane_allmin(x):
        for s in (64, 32, 16, 8, 4, 2, 1):
            x = jnp.minimum(x, pltpu.roll(x, s, 1))
        return x

    def body(_, carry):
        m3, col_kill, total = carry
        vv = jnp.reshape(m3, (8, 2, 8, _N))
        a, b = vv[:, 0], vv[:, 1]
        keep = a >= b
        pair = jax.lax.broadcasted_iota(jnp.int32, (8, 8, _N), 0)
        g = jnp.where(keep, pair * 2, pair * 2 + 1)
        v = jnp.where(keep, a, b)
        for n in (4, 2, 1):
            vv = jnp.reshape(v, (n, 2, 8, _N))
            gg = jnp.reshape(g, (n, 2, 8, _N))
            a, b = vv[:, 0], vv[:, 1]
            ga, gb = gg[:, 0], gg[:, 1]
            keep = a >= b
            v = jnp.where(keep, a, b)
            g = jnp.where(keep, ga, gb)
        v1 = v[0]
        row2_f = (g[0] * 8 + sub_iota).astype(jnp.float32)
        cm = jnp.max(v1, axis=0, keepdims=True)
        rmin_f = jnp.min(jnp.where(v1 == cm, row2_f, big_f),
                         axis=0, keepdims=True)
        cmm = cm + col_kill
        gmax = jnp.max(cmm, keepdims=True)
        key_f = jnp.min(jnp.where(cmm == gmax, rmin_f * _N + lane_f, big_f),
                        keepdims=True)
        key = key_f.astype(jnp.int32)
        g_r = jnp.reshape(jax.lax.shift_right_logical(key, 10), (1, 1, 1))
        s_r = jnp.reshape(jax.lax.shift_right_logical(key, 7) & 7, (1, 1, 1))
        c = key & (_N - 1)
        m3 = jnp.where((g_iota3 == g_r) & (s_iota3 == s_r), neg_inf, m3)
        col_kill = jnp.where(lane == c, neg_inf, col_kill)
        return m3, col_kill, total + gmax

    init = (m3_0, jnp.zeros((1, _N), jnp.float32),
            jnp.zeros((1, 1), jnp.float32))
    _, _, total = jax.lax.fori_loop(0, _N, body, init)
    out_ref[0, 0] = jnp.tanh(total[0, 0] / jnp.float32(_N))


def kernel(x1, x2):
    out = pl.pallas_call(
        _greedy_kernel,
        out_shape=jax.ShapeDtypeStruct((1, 1), jnp.float32),
        out_specs=pl.BlockSpec(memory_space=pltpu.SMEM),
    )(x1, x2)
    return jnp.reshape(out, (1,))


# speedup vs baseline: 67.4368x; 2.5699x over previous
import jax
import jax.numpy as jnp
from jax.experimental import pallas as pl
from jax.experimental.pallas import tpu as pltpu

_N = 128
_G = 16


def _greedy_kernel(x1_ref, x2_ref, out_ref):
    sub = jax.lax.dot_general(
        x1_ref[...], x2_ref[...],
        (((1,), (1,)), ((), ())),
        preferred_element_type=jnp.float32,
    )

    m3_0 = jnp.reshape(sub, (_G, 8, _N))
    lane = jax.lax.broadcasted_iota(jnp.int32, (1, _N), 1)
    lane_f = lane.astype(jnp.float32)
    sub_iota = jax.lax.broadcasted_iota(jnp.int32, (8, _N), 0)
    neg = jnp.float32(-1e30)
    live_thr = jnp.float32(-1e29)
    big_f = jnp.float32(1e9)
    r_iota = jax.lax.broadcasted_iota(jnp.int32, (_N, _N), 0)
    c_iota = jax.lax.broadcasted_iota(jnp.int32, (_N, _N), 1)
    eye = (r_iota == c_iota).astype(jnp.float32)
    r_col = jax.lax.broadcasted_iota(jnp.int32, (_N, 1), 0).astype(jnp.float32)
    ones_row = jnp.ones((1, _N), jnp.float32)
    ones_col = jnp.ones((_N, 1), jnp.float32)

    def cond(carry):
        return jnp.logical_and(carry[3] < _N, carry[4] < _N)

    def body(carry):
        m3, col_kill, total, cnt, ph = carry
        vv = jnp.reshape(m3, (8, 2, 8, _N))
        a, b = vv[:, 0], vv[:, 1]
        keep = a >= b
        pair = jax.lax.broadcasted_iota(jnp.int32, (8, 8, _N), 0)
        g = jnp.where(keep, pair * 2, pair * 2 + 1)
        v = jnp.where(keep, a, b)
        for n in (4, 2, 1):
            vvv = jnp.reshape(v, (n, 2, 8, _N))
            gg = jnp.reshape(g, (n, 2, 8, _N))
            a, b = vvv[:, 0], vvv[:, 1]
            ga, gb = gg[:, 0], gg[:, 1]
            keep = a >= b
            v = jnp.where(keep, a, b)
            g = jnp.where(keep, ga, gb)
        v1 = v[0]
        row2_f = (g[0] * 8 + sub_iota).astype(jnp.float32)
        cm = jnp.max(v1, axis=0, keepdims=True)
        rmin_f = jnp.min(jnp.where(v1 == cm, row2_f, big_f),
                         axis=0, keepdims=True)
        cmm = cm + col_kill
        keyv = rmin_f * _N + lane_f
        packed = jnp.concatenate([cmm, keyv, rmin_f], axis=0)
        packed_t = jax.lax.dot_general(
            eye, packed, (((1,), (1,)), ((), ())),
            preferred_element_type=jnp.float32,
            precision=jax.lax.Precision.HIGHEST)
        v_t = packed_t[:, 0:1]
        k_t = packed_t[:, 1:2]
        r_t = packed_t[:, 2:3]
        beats = jnp.logical_or(
            v_t > cmm, jnp.logical_and(v_t == cmm, k_t < keyv))
        beats_f = beats.astype(jnp.float32)
        dup_f = jnp.logical_and(beats, r_t == rmin_f).astype(jnp.float32)
        d_cnt = jax.lax.dot_general(
            ones_row, dup_f, (((1,), (0,)), ((), ())),
            preferred_element_type=jnp.float32)
        d_flag = (d_cnt >= 0.5).astype(jnp.float32)
        e_cnt = jax.lax.dot_general(
            d_flag, beats_f, (((1,), (0,)), ((), ())),
            preferred_element_type=jnp.float32)
        acc_b = jnp.logical_and(
            jnp.logical_and(e_cnt < 0.5, d_flag < 0.5), cmm > live_thr)
        acc = acc_b.astype(jnp.float32)
        rm_mat = (r_col == rmin_f).astype(jnp.float32)
        rowkill = jax.lax.dot_general(
            rm_mat, acc, (((1,), (1,)), ((), ())),
            preferred_element_type=jnp.float32)
        cnt_dot = jax.lax.dot_general(
            acc, ones_col, (((1,), (0,)), ((), ())),
            preferred_element_type=jnp.float32)
        m3 = jnp.where(jnp.reshape(rowkill, (_G, 8, 1)) >= 0.5, neg, m3)
        total = total + jnp.where(acc_b, cmm, 0.0)
        col_kill = jnp.where(acc_b, neg, col_kill)
        cnt = cnt + cnt_dot[0, 0].astype(jnp.int32)
        return (m3, col_kill, total, cnt, ph + 1)

    init = (m3_0, jnp.zeros((1, _N), jnp.float32),
            jnp.zeros((1, _N), jnp.float32), jnp.int32(0), jnp.int32(0))
    _, _, total, _, _ = jax.lax.while_loop(cond, body, init)
    out_ref[0, 0] = jnp.tanh(jnp.sum(total) / jnp.float32(_N))


def kernel(x1, x2):
    out = pl.pallas_call(
        _greedy_kernel,
        out_shape=jax.ShapeDtypeStruct((1, 1), jnp.float32),
        out_specs=pl.BlockSpec(memory_space=pltpu.SMEM),
    )(x1, x2)
    return jnp.reshape(out, (1,))


# speedup vs baseline: 88.7812x; 1.3165x over previous
import jax
import jax.numpy as jnp
from jax.experimental import pallas as pl
from jax.experimental.pallas import tpu as pltpu

_N = 128
_G = 16


def _greedy_kernel(x1_ref, x2_ref, out_ref):
    sub = jax.lax.dot_general(
        x1_ref[...], x2_ref[...],
        (((1,), (1,)), ((), ())),
        preferred_element_type=jnp.float32,
    )

    m3_0 = jnp.reshape(sub, (_G, 8, _N))
    lane = jax.lax.broadcasted_iota(jnp.int32, (1, _N), 1)
    lane_f = lane.astype(jnp.float32)
    sub_iota = jax.lax.broadcasted_iota(jnp.int32, (8, _N), 0)
    neg = jnp.float32(-1e30)
    live_thr = jnp.float32(-1e29)
    big_f = jnp.float32(1e9)
    int_min = jnp.int32(-(2**31 - 1) - 1)
    r_iota = jax.lax.broadcasted_iota(jnp.int32, (_N, _N), 0)
    c_iota = jax.lax.broadcasted_iota(jnp.int32, (_N, _N), 1)
    eye_b = r_iota == c_iota
    eye = eye_b.astype(jnp.float32)
    r_col = jax.lax.broadcasted_iota(jnp.int32, (_N, 1), 0)
    r_col_f = r_col.astype(jnp.float32)
    ones_row = jnp.ones((1, _N), jnp.float32)
    ones_col = jnp.ones((_N, 1), jnp.float32)

    def sortable(x):
        b = jax.lax.bitcast_convert_type(x + 0.0, jnp.int32)
        return b ^ (jax.lax.shift_right_arithmetic(b, 31)
                    & jnp.int32(0x7FFFFFFF))

    def split8(s):
        u = s ^ jnp.int32(-(2**31))
        return [(jax.lax.shift_right_logical(u, sh) & 0xFF)
                .astype(jnp.float32) for sh in (24, 16, 8, 0)]

    def join8(b3, b2, b1, b0):
        h = (b3 * 256 + b2).astype(jnp.int32)
        low = (b1 * 256 + b0).astype(jnp.int32)
        return (h - 32768) * 65536 + low

    def cond(carry):
        return jnp.logical_and(carry[3] < _N, carry[4] < _N)

    def body(carry):
        m3, col_kill, total, cnt, ph = carry
        vv = jnp.reshape(m3, (8, 2, 8, _N))
        a, b = vv[:, 0], vv[:, 1]
        keep = a >= b
        pair = jax.lax.broadcasted_iota(jnp.int32, (8, 8, _N), 0)
        g = jnp.where(keep, pair * 2, pair * 2 + 1)
        v = jnp.where(keep, a, b)
        w = jnp.minimum(a, b)
        for n in (4, 2, 1):
            vvv = jnp.reshape(v, (n, 2, 8, _N))
            gg = jnp.reshape(g, (n, 2, 8, _N))
            ww = jnp.reshape(w, (n, 2, 8, _N))
            a, b = vvv[:, 0], vvv[:, 1]
            ga, gb = gg[:, 0], gg[:, 1]
            wa, wb = ww[:, 0], ww[:, 1]
            keep = a >= b
            w = jnp.maximum(jnp.minimum(a, b), jnp.where(keep, wa, wb))
            v = jnp.where(keep, a, b)
            g = jnp.where(keep, ga, gb)
        v1 = v[0]
        w1 = w[0]
        row2_f = (g[0] * 8 + sub_iota).astype(jnp.float32)
        cm = jnp.max(v1, axis=0, keepdims=True)
        rmin_f = jnp.min(jnp.where(v1 == cm, row2_f, big_f),
                         axis=0, keepdims=True)
        win_cell = jnp.logical_and(v1 == cm, row2_f == rmin_f)
        v2 = jnp.max(jnp.where(win_cell, w1, v1), axis=0, keepdims=True)
        cmm = cm + col_kill
        srt = sortable(cmm)
        srt2 = sortable(v2)
        comps = split8(srt) + split8(srt2) + [rmin_f]
        packed = jnp.concatenate(comps, axis=0)
        packed_t = jax.lax.dot_general(
            eye, packed, (((1,), (1,)), ((), ())),
            preferred_element_type=jnp.float32)
        s_t = join8(packed_t[:, 0:1], packed_t[:, 1:2],
                    packed_t[:, 2:3], packed_t[:, 3:4])
        s2_t = join8(packed_t[:, 4:5], packed_t[:, 5:6],
                     packed_t[:, 6:7], packed_t[:, 7:8])
        r_t = packed_t[:, 8:9]
        key_lt = jnp.logical_or(
            r_t < rmin_f,
            jnp.logical_and(r_t == rmin_f, r_col_f < lane_f))
        beats = jnp.logical_or(
            s_t > srt, jnp.logical_and(s_t == srt, key_lt))
        same_row = r_t == rmin_f
        dupmat = jnp.logical_and(beats, same_row)
        dup = jnp.max(dupmat.astype(jnp.float32), axis=0, keepdims=True)
        dup_b = dup >= 0.5
        dupmat2 = jnp.logical_and(
            jnp.logical_not(jnp.logical_or(beats, eye_b)), same_row)
        dup_t = jax.lax.dot_general(
            dupmat2.astype(jnp.float32), ones_col, (((1,), (0,)), ((), ())),
            preferred_element_type=jnp.float32)
        thr = jnp.max(
            jnp.where(jnp.logical_and(dup_t >= 0.5, beats), s2_t, int_min),
            axis=0, keepdims=True)
        uncertain = jnp.logical_and(jnp.logical_not(dup_b), srt <= thr)
        stopped = jax.lax.dot_general(
            uncertain.astype(jnp.float32),
            jnp.logical_or(beats, eye_b).astype(jnp.float32),
            (((1,), (0,)), ((), ())),
            preferred_element_type=jnp.float32)
        acc_b = jnp.logical_and(
            jnp.logical_and(stopped < 0.5, jnp.logical_not(dup_b)),
            cmm > live_thr)
        acc = acc_b.astype(jnp.float32)
        rm_mat = (r_col_f == rmin_f).astype(jnp.float32)
        rowkill = jax.lax.dot_general(
            rm_mat, acc, (((1,), (1,)), ((), ())),
            preferred_element_type=jnp.float32)
        cnt_dot = jax.lax.dot_general(
            acc, ones_col, (((1,), (0,)), ((), ())),
            preferred_element_type=jnp.float32)
        m3 = jnp.where(jnp.reshape(rowkill, (_G, 8, 1)) >= 0.5, neg, m3)
        total = total + jnp.where(acc_b, cmm, 0.0)
        col_kill = jnp.where(acc_b, neg, col_kill)
        cnt = cnt + cnt_dot[0, 0].astype(jnp.int32)
        return (m3, col_kill, total, cnt, ph + 1)

    init = (m3_0, jnp.zeros((1, _N), jnp.float32),
            jnp.zeros((1, _N), jnp.float32), jnp.int32(0), jnp.int32(0))
    _, _, total, _, _ = jax.lax.while_loop(cond, body, init)
    out_ref[0, 0] = jnp.tanh(jnp.sum(total) / jnp.float32(_N))


def kernel(x1, x2):
    out = pl.pallas_call(
        _greedy_kernel,
        out_shape=jax.ShapeDtypeStruct((1, 1), jnp.float32),
        out_specs=pl.BlockSpec(memory_space=pltpu.SMEM),
    )(x1, x2)
    return jnp.reshape(out, (1,))
